# Initial kernel scaffold; baseline (speedup 1.0000x reference)
import jax
import jax.numpy as jnp
from jax.experimental import pallas as pl


def kernel(bbox_mu, bbox_log_sigma, pos_points, pos_strides, gt_centers, gt_centers_list, cur_iter):
    raise NotImplementedError("write your pallas kernel here")


# speedup vs baseline: 2.9035x; 2.9035x over previous
import functools

import jax
import jax.numpy as jnp
from jax.experimental import pallas as pl

_N = 10000
_M = 5000
_LAMBDA_CENTER = 1.0
_LAMBDA_KL = 0.05
_LAMBDA_KL_WARMUP = 0.005
_KNN_K = 5
_SIGMA_S_INIT = 2.0
_SIGMA_S_FINAL = 0.8
_WARMUP_ITERS = 1000
_ANNEAL_ITERS = 3000
_PRIOR_DELTA_MIN = 0.5
_PRIOR_DELTA_MAX = 20.0
_LOG_SIGMA_MIN = -6.0
_LOG_SIGMA_MAX = 4.0
_BIG = 3.0e38


def _body(mu_ref, bls_ref, pos_ref, stride_ref, gtc_ref, kx_ref, ky_ref,
          sig_ref, out_ref, *, m, k, n_valid):
    i = pl.program_id(0)

    q = gtc_ref[...]
    qx = q[:, 0:1]
    qy = q[:, 1:2]
    kx = kx_ref[...]
    ky = ky_ref[...]

    qn = qx * qx + qy * qy
    kn = kx * kx + ky * ky
    cross = qx * kx + qy * ky
    d2 = qn + kn - 2.0 * cross
    d = jnp.sqrt(jnp.clip(d2, 1e-12, None))
    d = jnp.where(d < 0.01, d + 1.0e8, d)

    colid = jax.lax.broadcasted_iota(jnp.int32, d.shape, 1)
    total = jnp.zeros((d.shape[0], 1), jnp.float32)
    for _ in range(k):
        mn = jnp.min(d, axis=1, keepdims=True)
        total = total + mn
        ism = d == mn
        first = jnp.min(jnp.where(ism, colid, m), axis=1, keepdims=True)
        d = jnp.where(colid == first, _BIG, d)
    d_i = total * (1.0 / k)

    stride = stride_ref[...]
    mu = mu_ref[...]
    pos = pos_ref[...]

    gt_delta = (q - pos) / stride
    diff = mu[:, 0:2] - gt_delta
    ad = jnp.abs(diff)
    sl1 = jnp.where(ad < 1.0, 0.5 * diff * diff, ad - 0.5)

    d_norm = jnp.clip(d_i / stride, _PRIOR_DELTA_MIN, _PRIOR_DELTA_MAX)
    sigma_c = jnp.maximum(d_norm, 1.0)
    mu_s = jnp.log(d_norm)
    sig_s = sig_ref[0, 0]

    log_sq = jnp.clip(bls_ref[...], _LOG_SIGMA_MIN, _LOG_SIGMA_MAX)
    sigma_q = jnp.exp(log_sq)

    prior_mu = jnp.concatenate(
        [jnp.zeros_like(d_norm), jnp.zeros_like(d_norm), mu_s, mu_s], axis=1)
    prior_sigma = jnp.concatenate(
        [sigma_c, sigma_c,
         jnp.full_like(d_norm, 1.0) * sig_s,
         jnp.full_like(d_norm, 1.0) * sig_s], axis=1)
    sigma_p = jnp.clip(prior_sigma, 0.0001, None)

    dm = mu - prior_mu
    kl = (jnp.log(sigma_p / sigma_q)
          + (sigma_q * sigma_q + dm * dm) / (2.0 * sigma_p * sigma_p) - 0.5)

    br = q.shape[0]
    rowid = i * br + jax.lax.broadcasted_iota(jnp.int32, (br, 1), 0)
    rmask = rowid < n_valid
    s_center = jnp.sum(jnp.where(rmask, sl1[:, 0:1] + sl1[:, 1:2], 0.0))
    s_ckl = jnp.sum(jnp.where(rmask, kl[:, 0:1] + kl[:, 1:2], 0.0))
    s_skl = jnp.sum(jnp.where(rmask, kl[:, 2:3] + kl[:, 3:4], 0.0))

    lane = jax.lax.broadcasted_iota(jnp.int32, (1, 128), 1)
    vec = (jnp.where(lane == 0, s_center, 0.0)
           + jnp.where(lane == 1, s_ckl, 0.0)
           + jnp.where(lane == 2, s_skl, 0.0))

    @pl.when(i == 0)
    def _():
        out_ref[...] = jnp.zeros_like(out_ref)

    out_ref[...] += vec


def _pick_br(n):
    for br in (400, 256, 128, 64, 32, 16, 8):
        if n % br == 0:
            return br
    return n


@jax.jit
def kernel(bbox_mu, bbox_log_sigma, pos_points, pos_strides, gt_centers,
           gt_centers_list, cur_iter):
    n = bbox_mu.shape[0]
    keys = gt_centers_list.reshape(-1, 2)
    m = keys.shape[0]
    k = min(_KNN_K, m - 1)

    ratio = jnp.clip((cur_iter - _WARMUP_ITERS) / max(_ANNEAL_ITERS, 1), 0.0, 1.0)
    eff_lkl = _LAMBDA_KL_WARMUP + ratio * (_LAMBDA_KL - _LAMBDA_KL_WARMUP)
    sigma_s = _SIGMA_S_INIT - ratio * (_SIGMA_S_INIT - _SIGMA_S_FINAL)
    sig_eff = jnp.maximum(sigma_s, 1.0).astype(jnp.float32).reshape(1, 1)

    br = _pick_br(n)
    nb = n // br

    kxT = keys[:, 0].reshape(1, m)
    kyT = keys[:, 1].reshape(1, m)
    stride2d = pos_strides.astype(jnp.float32).reshape(n, 1)

    row_spec = lambda c: pl.BlockSpec((br, c), lambda i: (i, 0))
    full_spec = lambda r, c: pl.BlockSpec((r, c), lambda i: (0, 0))

    out = pl.pallas_call(
        functools.partial(_body, m=m, k=k, n_valid=n),
        grid=(nb,),
        in_specs=[
            row_spec(4),
            row_spec(4),
            row_spec(2),
            row_spec(1),
            row_spec(2),
            full_spec(1, m),
            full_spec(1, m),
            full_spec(1, 1),
        ],
        out_specs=pl.BlockSpec((1, 128), lambda i: (0, 0)),
        out_shape=jax.ShapeDtypeStruct((1, 128), jnp.float32),
    )(bbox_mu, bbox_log_sigma, pos_points, stride2d, gt_centers,
      kxT, kyT, sig_eff)

    s_center = out[0, 0]
    s_ckl = out[0, 1]
    s_skl = out[0, 2]

    l_center = s_center / n
    center_kl = s_ckl / n
    scale_kl = s_skl / n
    l_kl = center_kl + ratio * scale_kl
    weighted_center = (_LAMBDA_CENTER * l_center).astype(jnp.float32)
    weighted_kl = (eff_lkl * l_kl).astype(jnp.float32)
    return (weighted_center, weighted_kl)
